# baseline (speedup 1.0000x reference)
import functools

import jax
import jax.numpy as jnp
from jax import lax
from jax.experimental import pallas as pl
from jax.experimental.pallas import tpu as pltpu
from jax.experimental.pallas import tpu_sc as plsc

_DIM = 1024
_ROWS = 8192
_SCALE = _DIM ** (-0.5)

_NC = 2
_NS = 16
_L = 16
_NW = _NC * _NS

_ELEMS = _ROWS * _DIM
_EPW = _ELEMS // _NW
_CHUNK = 32768
_NCHUNKS = _EPW // _CHUNK
_VECS = _CHUNK // _L
_UNROLL = 8

_mesh = plsc.VectorSubcoreMesh(core_axis_name="c", subcore_axis_name="s")


@functools.partial(
    pl.kernel,
    mesh=_mesh,
    out_type=jax.ShapeDtypeStruct((_ELEMS,), jnp.float32),
    scratch_types=[
        pltpu.VMEM((_CHUNK,), jnp.float32),
        pltpu.SemaphoreType.DMA,
    ],
)
def _scale_sc(emb_hbm, out_hbm, buf, sem):
    wid = lax.axis_index("s") * _NC + lax.axis_index("c")
    base = wid * _EPW

    def chunk_body(g, carry):
        off = base + g * _CHUNK
        pltpu.async_copy(emb_hbm.at[pl.ds(off, _CHUNK)], buf, sem).wait()

        def scale_body(j, c):
            b0 = j * (_L * _UNROLL)
            for u in range(_UNROLL):
                sl = pl.ds(b0 + u * _L, _L)
                buf[sl] = buf[sl] * _SCALE
            return c

        lax.fori_loop(0, _VECS // _UNROLL, scale_body, 0)
        pltpu.async_copy(buf, out_hbm.at[pl.ds(off, _CHUNK)], sem).wait()
        return carry

    lax.fori_loop(0, _NCHUNKS, chunk_body, 0)


def kernel(x, emb):
    del x
    out_flat = _scale_sc(emb.reshape(_ELEMS))
    return out_flat.reshape(_ROWS, _DIM)


# speedup vs baseline: 1.0067x; 1.0067x over previous
import functools

import jax
import jax.numpy as jnp
from jax import lax
from jax.experimental import pallas as pl
from jax.experimental.pallas import tpu as pltpu
from jax.experimental.pallas import tpu_sc as plsc

_DIM = 1024
_ROWS = 8192
_SCALE = _DIM ** (-0.5)

_NC = 2
_NS = 16
_L = 16
_NW = _NC * _NS

_ELEMS = _ROWS * _DIM
_EPW = _ELEMS // _NW
_CHUNK = 32768
_NCHUNKS = _EPW // _CHUNK
_VECS = _CHUNK // _L
_UNROLL = 8

_mesh = plsc.VectorSubcoreMesh(core_axis_name="c", subcore_axis_name="s")


@functools.partial(
    pl.kernel,
    mesh=_mesh,
    out_type=jax.ShapeDtypeStruct((_ELEMS,), jnp.float32),
    scratch_types=[
        pltpu.VMEM((_CHUNK,), jnp.float32),
        pltpu.SemaphoreType.DMA,
    ],
)
def _scale_sc(emb_hbm, out_hbm, buf, sem):
    wid = lax.axis_index("s") * _NC + lax.axis_index("c")
    base = wid * _EPW

    def chunk_body(g, carry):
        off = base + g * _CHUNK
        pltpu.async_copy(emb_hbm.at[pl.ds(off, _CHUNK)], buf, sem).wait()

        @plsc.parallel_loop(0, _CHUNK, step=_L, unroll=_UNROLL)
        def scale_body(i):
            sl = pl.ds(i, _L)
            buf[sl] = buf[sl] * _SCALE

        pltpu.async_copy(buf, out_hbm.at[pl.ds(off, _CHUNK)], sem).wait()
        return carry

    lax.fori_loop(0, _NCHUNKS, chunk_body, 0)


def kernel(x, emb):
    del x
    out_flat = _scale_sc(emb.reshape(_ELEMS))
    return out_flat.reshape(_ROWS, _DIM)


# speedup vs baseline: 1.9452x; 1.9323x over previous
import functools

import jax
import jax.numpy as jnp
from jax import lax
from jax.experimental import pallas as pl
from jax.experimental.pallas import tpu as pltpu
from jax.experimental.pallas import tpu_sc as plsc

_DIM = 1024
_ROWS = 8192
_SCALE = _DIM ** (-0.5)

_NC = 2
_NS = 16
_L = 16
_NW = _NC * _NS

_ROWS_PER_W = _ROWS // _NW
_CHUNK_ROWS = 32
_NCHUNKS = _ROWS_PER_W // _CHUNK_ROWS
_VECS_PER_ROW = _DIM // _L

_mesh = plsc.VectorSubcoreMesh(core_axis_name="c", subcore_axis_name="s")


@functools.partial(
    pl.kernel,
    mesh=_mesh,
    out_type=jax.ShapeDtypeStruct((_ROWS, _DIM), jnp.float32),
    scratch_types=[
        pltpu.VMEM((_CHUNK_ROWS, _DIM), jnp.float32),
        pltpu.SemaphoreType.DMA,
    ],
    compiler_params=pltpu.CompilerParams(use_tc_tiling_on_sc=True),
)
def _scale_sc(emb_hbm, out_hbm, buf, sem):
    wid = lax.axis_index("s") * _NC + lax.axis_index("c")
    row0 = wid * _ROWS_PER_W

    def chunk_body(g, carry):
        r0 = row0 + g * _CHUNK_ROWS
        pltpu.async_copy(emb_hbm.at[pl.ds(r0, _CHUNK_ROWS)], buf, sem).wait()

        @plsc.parallel_loop(0, _CHUNK_ROWS, step=1, unroll=2)
        def scale_row(r):
            for c in range(_VECS_PER_ROW):
                sl = pl.ds(c * _L, _L)
                buf[r, sl] = buf[r, sl] * _SCALE

        pltpu.async_copy(buf, out_hbm.at[pl.ds(r0, _CHUNK_ROWS)], sem).wait()
        return carry

    lax.fori_loop(0, _NCHUNKS, chunk_body, 0)


def kernel(x, emb):
    del x
    return _scale_sc(emb)


# speedup vs baseline: 2.1753x; 1.1183x over previous
import functools

import jax
import jax.numpy as jnp
from jax import lax
from jax.experimental import pallas as pl
from jax.experimental.pallas import tpu as pltpu
from jax.experimental.pallas import tpu_sc as plsc

_DIM = 1024
_ROWS = 8192
_SCALE = _DIM ** (-0.5)

_NC = 2
_NS = 16
_L = 16
_NW = _NC * _NS

_ROWS_PER_W = _ROWS // _NW
_CHUNK_ROWS = 32
_NCHUNKS = _ROWS_PER_W // _CHUNK_ROWS
_VECS_PER_ROW = _DIM // _L

_mesh = plsc.VectorSubcoreMesh(core_axis_name="c", subcore_axis_name="s")


_NBUF = 3


@functools.partial(
    pl.kernel,
    mesh=_mesh,
    out_type=jax.ShapeDtypeStruct((_ROWS, _DIM), jnp.float32),
    scratch_types=[
        pltpu.VMEM((_NBUF, _CHUNK_ROWS, _DIM), jnp.float32),
        pltpu.SemaphoreType.DMA((_NBUF,)),
        pltpu.SemaphoreType.DMA((_NBUF,)),
    ],
    compiler_params=pltpu.CompilerParams(use_tc_tiling_on_sc=True),
)
def _scale_sc(emb_hbm, out_hbm, buf, in_sems, out_sems):
    wid = lax.axis_index("s") * _NC + lax.axis_index("c")
    row0 = wid * _ROWS_PER_W

    def start_in(g):
        b = g % _NBUF
        return pltpu.async_copy(
            emb_hbm.at[pl.ds(row0 + g * _CHUNK_ROWS, _CHUNK_ROWS)],
            buf.at[b], in_sems.at[b])

    def start_out(g):
        b = g % _NBUF
        return pltpu.async_copy(
            buf.at[b],
            out_hbm.at[pl.ds(row0 + g * _CHUNK_ROWS, _CHUNK_ROWS)],
            out_sems.at[b])

    def scale_chunk(b):
        @plsc.parallel_loop(0, _CHUNK_ROWS, step=1, unroll=2)
        def scale_row(r):
            for c in range(_VECS_PER_ROW):
                sl = pl.ds(c * _L, _L)
                buf[b, r, sl] = buf[b, r, sl] * _SCALE

    in_h = {}
    out_h = {}
    in_h[0] = start_in(0)
    in_h[1] = start_in(1)
    for g in range(_NCHUNKS):
        in_h[g].wait()
        scale_chunk(g % _NBUF)
        out_h[g] = start_out(g)
        nxt = g + 2
        if nxt < _NCHUNKS:
            if nxt - _NBUF >= 0:
                out_h[nxt - _NBUF].wait()
            in_h[nxt] = start_in(nxt)
    for g in range(_NCHUNKS - _NBUF, _NCHUNKS):
        if g >= 0:
            out_h[g].wait()


def kernel(x, emb):
    del x
    return _scale_sc(emb)


# speedup vs baseline: 2.6889x; 1.2361x over previous
import functools

import jax
import jax.numpy as jnp
from jax import lax
from jax.experimental import pallas as pl
from jax.experimental.pallas import tpu as pltpu
from jax.experimental.pallas import tpu_sc as plsc

_DIM = 1024
_ROWS = 8192
_SCALE = _DIM ** (-0.5)

_NC = 2
_NS = 16
_L = 16
_NW = _NC * _NS

_ROWS_PER_W = _ROWS // _NW
_CHUNK_ROWS = 32
_NCHUNKS = _ROWS_PER_W // _CHUNK_ROWS
_VECS_PER_ROW = _DIM // _L

_mesh = plsc.VectorSubcoreMesh(core_axis_name="c", subcore_axis_name="s")


_NBUF = 3


@functools.partial(
    pl.kernel,
    mesh=_mesh,
    out_type=jax.ShapeDtypeStruct((_ROWS, _DIM), jnp.float32),
    scratch_types=[
        pltpu.VMEM((_NBUF, _CHUNK_ROWS, _DIM), jnp.float32),
        pltpu.SemaphoreType.DMA((_NBUF,)),
        pltpu.SemaphoreType.DMA((_NBUF,)),
    ],
    compiler_params=pltpu.CompilerParams(use_tc_tiling_on_sc=True),
)
def _scale_sc(emb_hbm, out_hbm, buf, in_sems, out_sems):
    wid = lax.axis_index("s") * _NC + lax.axis_index("c")
    row0 = wid * _ROWS_PER_W

    def in_copy(g, b):
        return pltpu.make_async_copy(
            emb_hbm.at[pl.ds(row0 + g * _CHUNK_ROWS, _CHUNK_ROWS)],
            buf.at[b], in_sems.at[b])

    def out_copy(g, b):
        return pltpu.make_async_copy(
            buf.at[b],
            out_hbm.at[pl.ds(row0 + g * _CHUNK_ROWS, _CHUNK_ROWS)],
            out_sems.at[b])

    in_copy(0, 0).start()
    in_copy(1, 1).start()

    def chunk_body(g, carry):
        b = g % _NBUF
        in_copy(g, b).wait()

        @plsc.parallel_loop(0, _CHUNK_ROWS, step=1, unroll=2)
        def scale_row(r):
            for c in range(_VECS_PER_ROW):
                sl = pl.ds(c * _L, _L)
                buf[b, r, sl] = buf[b, r, sl] * _SCALE

        out_copy(g, b).start()

        @pl.when(g + 2 < _NCHUNKS)
        def _prefetch():
            b2 = (g + 2) % _NBUF

            @pl.when(g >= 1)
            def _drain():
                out_copy(g - 1, b2).wait()

            in_copy(g + 2, b2).start()

        return carry

    lax.fori_loop(0, _NCHUNKS, chunk_body, 0)
    for g in range(_NCHUNKS - _NBUF, _NCHUNKS):
        if g >= 0:
            out_copy(g, g % _NBUF).wait()


def kernel(x, emb):
    del x
    return _scale_sc(emb)


# speedup vs baseline: 2.7148x; 1.0096x over previous
import functools

import jax
import jax.numpy as jnp
from jax import lax
from jax.experimental import pallas as pl
from jax.experimental.pallas import tpu as pltpu
from jax.experimental.pallas import tpu_sc as plsc

_DIM = 1024
_ROWS = 8192
_SCALE = _DIM ** (-0.5)

_NC = 2
_NS = 16
_L = 16
_NW = _NC * _NS

_ROWS_PER_W = _ROWS // _NW
_CHUNK_ROWS = 32
_NCHUNKS = _ROWS_PER_W // _CHUNK_ROWS
_VECS_PER_ROW = _DIM // _L

_mesh = plsc.VectorSubcoreMesh(core_axis_name="c", subcore_axis_name="s")


_NBUF = 3


@functools.partial(
    pl.kernel,
    mesh=_mesh,
    out_type=jax.ShapeDtypeStruct((_ROWS, _DIM), jnp.float32),
    scratch_types=[
        pltpu.VMEM((_NBUF, _CHUNK_ROWS, _DIM), jnp.float32),
        pltpu.SemaphoreType.DMA((_NBUF,)),
        pltpu.SemaphoreType.DMA((_NBUF,)),
    ],
    compiler_params=pltpu.CompilerParams(use_tc_tiling_on_sc=True),
)
def _scale_sc(emb_hbm, out_hbm, buf, in_sems, out_sems):
    wid = lax.axis_index("s") * _NC + lax.axis_index("c")
    row0 = wid * _ROWS_PER_W

    def in_copy(g, b):
        return pltpu.make_async_copy(
            emb_hbm.at[pl.ds(row0 + g * _CHUNK_ROWS, _CHUNK_ROWS)],
            buf.at[b], in_sems.at[b])

    def out_copy(g, b):
        return pltpu.make_async_copy(
            buf.at[b],
            out_hbm.at[pl.ds(row0 + g * _CHUNK_ROWS, _CHUNK_ROWS)],
            out_sems.at[b])

    in_copy(0, 0).start()
    in_copy(1, 1).start()

    def chunk_body(g, carry):
        b = g % _NBUF
        in_copy(g, b).wait()

        @pl.when(g + 2 < _NCHUNKS)
        def _prefetch():
            b2 = (g + 2) % _NBUF

            @pl.when(g >= 1)
            def _drain():
                out_copy(g - 1, b2).wait()

            in_copy(g + 2, b2).start()

        @plsc.parallel_loop(0, _CHUNK_ROWS, step=1, unroll=4)
        def scale_row(r):
            for c in range(_VECS_PER_ROW):
                sl = pl.ds(c * _L, _L)
                buf[b, r, sl] = buf[b, r, sl] * _SCALE

        out_copy(g, b).start()
        return carry

    lax.fori_loop(0, _NCHUNKS, chunk_body, 0)
    for g in range(_NCHUNKS - _NBUF, _NCHUNKS):
        if g >= 0:
            out_copy(g, g % _NBUF).wait()


def kernel(x, emb):
    del x
    return _scale_sc(emb)
